# baseline (speedup 1.0000x reference)
import functools

import jax
import jax.numpy as jnp
from jax.experimental import pallas as pl

B = 1024
N_ENT = 64
F = 64
H = 256
N_FOCUS = 32
DS_OUT = 260
BB = 64
NB = B // BB


def _mlp(x, W1, b1, W2, b2):
    h = jnp.maximum(jnp.dot(x, W1, preferred_element_type=jnp.float32) + b1, 0.0)
    return jnp.maximum(jnp.dot(h, W2, preferred_element_type=jnp.float32) + b2, 0.0)


def _softmax(x):
    m = jnp.max(x, axis=-1, keepdims=True)
    unnorm = jnp.exp(x - m)
    return unnorm / jnp.sum(unnorm, axis=-1, keepdims=True)


G = 4


def _ranks(p):
    pm = p[:, None, :]
    pn = p[:, :, None]
    im = jax.lax.broadcasted_iota(jnp.int32, (BB, N_ENT, N_ENT), 2)
    i_n = jax.lax.broadcasted_iota(jnp.int32, (BB, N_ENT, N_ENT), 1)
    beats = (pm > pn) | ((pm == pn) & (im < i_n))
    return jnp.sum(beats.astype(jnp.int32), axis=-1)


def _prune(rank, ve):
    j_iota = jax.lax.broadcasted_iota(jnp.int32, (BB, N_FOCUS, N_ENT), 1)
    P3 = (rank[:, None, :] == j_iota).astype(jnp.float32)
    P3f = P3.reshape(BB * N_FOCUS, N_ENT)
    r_io = jax.lax.broadcasted_iota(jnp.int32, (G * N_FOCUS, G * N_ENT), 0)
    c_io = jax.lax.broadcasted_iota(jnp.int32, (G * N_FOCUS, G * N_ENT), 1)
    bd = ((r_io // N_FOCUS) == (c_io // N_ENT)).astype(jnp.float32)
    veh = ve
    chunks = []
    for g in range(BB // G):
        Pg = P3f[G * N_FOCUS * g:G * N_FOCUS * (g + 1), :]
        Pt = jnp.concatenate([Pg] * G, axis=1)
        ve_g = veh[G * N_ENT * g:G * N_ENT * (g + 1), :]
        chunks.append(jnp.dot(Pt * bd, ve_g, preferred_element_type=jnp.float32))
    return jnp.concatenate(chunks, axis=0)


def _sampo_kernel(ally_ref, enemy_ref, self_ref,
                  oW1_ref, ob1_ref, oW2_ref, ob2_ref,
                  sW1_ref, sb1_ref, sW2_ref, sb2_ref,
                  W1a_ref, W1e_ref, W1s_ref, db1_ref, dW2_ref, db2_ref,
                  Wq_a_ref, Wk_a_ref, Wq_e_ref, Wk_e_ref,
                  out_ref):
    norm = 0.0625
    ve_a = _mlp(ally_ref[...], oW1_ref[...], ob1_ref[...], oW2_ref[...], ob2_ref[...])
    ve_e = _mlp(enemy_ref[...], oW1_ref[...], ob1_ref[...], oW2_ref[...], ob2_ref[...])
    vs = _mlp(self_ref[...], sW1_ref[...], sb1_ref[...], sW2_ref[...], sb2_ref[...])

    hidden = jnp.dot(vs, W1s_ref[...], preferred_element_type=jnp.float32) + db1_ref[...]
    for ve, Wq_ref, Wk_ref, W1_ref in ((ve_a, Wq_a_ref, Wk_a_ref, W1a_ref),
                                       (ve_e, Wq_e_ref, Wk_e_ref, W1e_ref)):
        Q = jnp.dot(vs, Wq_ref[...], preferred_element_type=jnp.float32)
        K = jnp.dot(ve, Wk_ref[...], preferred_element_type=jnp.float32)
        K3 = K.reshape(BB, N_ENT, H)
        compat = norm * jax.lax.dot_general(
            Q[:, None, :], K3, (((2,), (2,)), ((0,), (0,))),
            preferred_element_type=jnp.float32)[:, 0, :]
        p = _softmax(compat)
        pruned = _prune(_ranks(p), ve)
        hidden = hidden + jnp.dot(
            pruned.reshape(BB, N_FOCUS * H), W1_ref[...],
            preferred_element_type=jnp.float32)
    hidden = jnp.maximum(hidden, 0.0)
    out = jnp.maximum(
        jnp.dot(hidden, dW2_ref[...], preferred_element_type=jnp.float32) + db2_ref[...],
        0.0)
    out_ref[...] = out


@functools.partial(jax.jit, static_argnames=())
def kernel(inp, other_W1, other_b1, other_W2, other_b2,
           self_W1, self_b1, self_W2, self_b2,
           ds_W1, ds_b1, ds_W2, ds_b2,
           Wq_a, Wk_a, Wq_e, Wk_e):
    a_sz = N_ENT * F
    ally_x = inp[:, :a_sz].reshape(B * N_ENT, F)
    enemy_x = inp[:, a_sz:2 * a_sz].reshape(B * N_ENT, F)
    self_x = inp[:, 2 * a_sz:2 * a_sz + F]
    W1a = ds_W1[:N_FOCUS * H]
    W1e = ds_W1[N_FOCUS * H:2 * N_FOCUS * H]
    W1s = ds_W1[2 * N_FOCUS * H:]

    row = lambda v: v.reshape(1, -1)
    wspec = lambda arr: pl.BlockSpec(arr.shape, lambda i: (0,) * arr.ndim)

    weights = [other_W1, row(other_b1), other_W2, row(other_b2),
               self_W1, row(self_b1), self_W2, row(self_b2),
               W1a, W1e, W1s, row(ds_b1), ds_W2, row(ds_b2),
               Wq_a, Wk_a, Wq_e, Wk_e]

    return pl.pallas_call(
        _sampo_kernel,
        grid=(NB,),
        in_specs=[
            pl.BlockSpec((BB * N_ENT, F), lambda i: (i, 0)),
            pl.BlockSpec((BB * N_ENT, F), lambda i: (i, 0)),
            pl.BlockSpec((BB, F), lambda i: (i, 0)),
        ] + [wspec(w) for w in weights],
        out_specs=pl.BlockSpec((BB, DS_OUT), lambda i: (i, 0)),
        out_shape=jax.ShapeDtypeStruct((B, DS_OUT), jnp.float32),
    )(ally_x, enemy_x, self_x, *weights)


# speedup vs baseline: 1.2478x; 1.2478x over previous
import functools

import jax
import jax.numpy as jnp
from jax.experimental import pallas as pl

B = 1024
N_ENT = 64
F = 64
H = 256
N_FOCUS = 32
DS_OUT = 260
BB = 64
NB = B // BB


def _mlp(x, W1, b1, W2, b2):
    h = jnp.maximum(jnp.dot(x, W1, preferred_element_type=jnp.float32) + b1, 0.0)
    return jnp.maximum(jnp.dot(h, W2, preferred_element_type=jnp.float32) + b2, 0.0)


def _softmax(x):
    m = jnp.max(x, axis=-1, keepdims=True)
    unnorm = jnp.exp(x - m)
    return unnorm / jnp.sum(unnorm, axis=-1, keepdims=True)


G = 4


def _ranks(p):
    pm = p[:, None, :]
    pn = p[:, :, None]
    im = jax.lax.broadcasted_iota(jnp.int32, (BB, N_ENT, N_ENT), 2)
    i_n = jax.lax.broadcasted_iota(jnp.int32, (BB, N_ENT, N_ENT), 1)
    beats = (pm > pn) | ((pm == pn) & (im < i_n))
    return jnp.sum(beats.astype(jnp.int32), axis=-1)


def _prune(rank, ve3):
    j_iota = jax.lax.broadcasted_iota(jnp.int32, (BB, N_FOCUS, N_ENT), 1)
    P = (rank[:, None, :] == j_iota).astype(jnp.float32)
    return jax.lax.dot_general(
        P, ve3, (((2,), (1,)), ((0,), (0,))),
        preferred_element_type=jnp.float32)


def _sampo_kernel(ally_ref, enemy_ref, self_ref,
                  oW1_ref, ob1_ref, oW2_ref, ob2_ref,
                  sW1_ref, sb1_ref, sW2_ref, sb2_ref,
                  W1a_ref, W1e_ref, W1s_ref, db1_ref, dW2_ref, db2_ref,
                  Wq_a_ref, Wk_a_ref, Wq_e_ref, Wk_e_ref,
                  out_ref):
    norm = 0.0625
    ve_a = _mlp(ally_ref[...], oW1_ref[...], ob1_ref[...], oW2_ref[...], ob2_ref[...])
    ve_e = _mlp(enemy_ref[...], oW1_ref[...], ob1_ref[...], oW2_ref[...], ob2_ref[...])
    vs = _mlp(self_ref[...], sW1_ref[...], sb1_ref[...], sW2_ref[...], sb2_ref[...])

    hidden = jnp.dot(vs, W1s_ref[...], preferred_element_type=jnp.float32) + db1_ref[...]
    for ve, Wq_ref, Wk_ref, W1_ref in ((ve_a, Wq_a_ref, Wk_a_ref, W1a_ref),
                                       (ve_e, Wq_e_ref, Wk_e_ref, W1e_ref)):
        Q = jnp.dot(vs, Wq_ref[...], preferred_element_type=jnp.float32)
        K = jnp.dot(ve, Wk_ref[...], preferred_element_type=jnp.float32)
        K3 = K.reshape(BB, N_ENT, H)
        compat = norm * jax.lax.dot_general(
            Q[:, None, :], K3, (((2,), (2,)), ((0,), (0,))),
            preferred_element_type=jnp.float32)[:, 0, :]
        p = _softmax(compat)
        pruned = _prune(_ranks(p), ve.reshape(BB, N_ENT, H))
        hidden = hidden + jnp.dot(
            pruned.reshape(BB, N_FOCUS * H), W1_ref[...],
            preferred_element_type=jnp.float32)
    hidden = jnp.maximum(hidden, 0.0)
    out = jnp.maximum(
        jnp.dot(hidden, dW2_ref[...], preferred_element_type=jnp.float32) + db2_ref[...],
        0.0)
    out_ref[...] = out


@functools.partial(jax.jit, static_argnames=())
def kernel(inp, other_W1, other_b1, other_W2, other_b2,
           self_W1, self_b1, self_W2, self_b2,
           ds_W1, ds_b1, ds_W2, ds_b2,
           Wq_a, Wk_a, Wq_e, Wk_e):
    a_sz = N_ENT * F
    ally_x = inp[:, :a_sz].reshape(B * N_ENT, F)
    enemy_x = inp[:, a_sz:2 * a_sz].reshape(B * N_ENT, F)
    self_x = inp[:, 2 * a_sz:2 * a_sz + F]
    W1a = ds_W1[:N_FOCUS * H]
    W1e = ds_W1[N_FOCUS * H:2 * N_FOCUS * H]
    W1s = ds_W1[2 * N_FOCUS * H:]

    row = lambda v: v.reshape(1, -1)
    wspec = lambda arr: pl.BlockSpec(arr.shape, lambda i: (0,) * arr.ndim)

    weights = [other_W1, row(other_b1), other_W2, row(other_b2),
               self_W1, row(self_b1), self_W2, row(self_b2),
               W1a, W1e, W1s, row(ds_b1), ds_W2, row(ds_b2),
               Wq_a, Wk_a, Wq_e, Wk_e]

    return pl.pallas_call(
        _sampo_kernel,
        grid=(NB,),
        in_specs=[
            pl.BlockSpec((BB * N_ENT, F), lambda i: (i, 0)),
            pl.BlockSpec((BB * N_ENT, F), lambda i: (i, 0)),
            pl.BlockSpec((BB, F), lambda i: (i, 0)),
        ] + [wspec(w) for w in weights],
        out_specs=pl.BlockSpec((BB, DS_OUT), lambda i: (i, 0)),
        out_shape=jax.ShapeDtypeStruct((B, DS_OUT), jnp.float32),
    )(ally_x, enemy_x, self_x, *weights)


# speedup vs baseline: 1.2680x; 1.0162x over previous
import functools

import jax
import jax.numpy as jnp
from jax.experimental import pallas as pl

B = 1024
N_ENT = 64
F = 64
H = 256
N_FOCUS = 32
DS_OUT = 260
BB = 64
NB = B // BB


def _mlp(x, W1, b1, W2, b2):
    h = jnp.maximum(jnp.dot(x, W1, preferred_element_type=jnp.float32) + b1, 0.0)
    return jnp.maximum(jnp.dot(h, W2, preferred_element_type=jnp.float32) + b2, 0.0)


def _softmax(x):
    m = jnp.max(x, axis=-1, keepdims=True)
    unnorm = jnp.exp(x - m)
    return unnorm / jnp.sum(unnorm, axis=-1, keepdims=True)


G = 4


def _ranks(p):
    pm = p[:, None, :]
    pn = p[:, :, None]
    im = jax.lax.broadcasted_iota(jnp.int32, (BB, N_ENT, N_ENT), 2)
    i_n = jax.lax.broadcasted_iota(jnp.int32, (BB, N_ENT, N_ENT), 1)
    beats = (pm > pn) | ((pm == pn) & (im < i_n))
    return jnp.sum(beats.astype(jnp.int32), axis=-1)


def _prune(rank, ve3):
    j_iota = jax.lax.broadcasted_iota(jnp.int32, (BB, N_FOCUS, N_ENT), 1)
    P = (rank[:, None, :] == j_iota).astype(jnp.float32)
    return jax.lax.dot_general(
        P, ve3, (((2,), (1,)), ((0,), (0,))),
        preferred_element_type=jnp.float32)


def _sampo_kernel(ally_ref, enemy_ref, self_ref,
                  oW1_ref, ob1_ref, oW2_ref, ob2_ref,
                  sW1_ref, sb1_ref, sW2_ref, sb2_ref,
                  dW1_ref, db1_ref, dW2_ref, db2_ref,
                  Wq_a_ref, Wk_a_ref, Wq_e_ref, Wk_e_ref,
                  out_ref):
    norm = 0.0625
    ve_a = _mlp(ally_ref[...], oW1_ref[...], ob1_ref[...], oW2_ref[...], ob2_ref[...])
    ve_e = _mlp(enemy_ref[...], oW1_ref[...], ob1_ref[...], oW2_ref[...], ob2_ref[...])
    vs = _mlp(self_ref[...], sW1_ref[...], sb1_ref[...], sW2_ref[...], sb2_ref[...])

    W1a = dW1_ref[:N_FOCUS * H, :]
    W1e = dW1_ref[N_FOCUS * H:2 * N_FOCUS * H, :]
    W1s = dW1_ref[2 * N_FOCUS * H:, :]
    hidden = jnp.dot(vs, W1s, preferred_element_type=jnp.float32) + db1_ref[...]
    for ve, Wq_ref, Wk_ref, W1 in ((ve_a, Wq_a_ref, Wk_a_ref, W1a),
                                   (ve_e, Wq_e_ref, Wk_e_ref, W1e)):
        Q = jnp.dot(vs, Wq_ref[...], preferred_element_type=jnp.float32)
        K = jnp.dot(ve, Wk_ref[...], preferred_element_type=jnp.float32)
        K3 = K.reshape(BB, N_ENT, H)
        compat = norm * jax.lax.dot_general(
            Q[:, None, :], K3, (((2,), (2,)), ((0,), (0,))),
            preferred_element_type=jnp.float32)[:, 0, :]
        p = _softmax(compat)
        pruned = _prune(_ranks(p), ve.reshape(BB, N_ENT, H))
        hidden = hidden + jnp.dot(
            pruned.reshape(BB, N_FOCUS * H), W1,
            preferred_element_type=jnp.float32)
    hidden = jnp.maximum(hidden, 0.0)
    out = jnp.maximum(
        jnp.dot(hidden, dW2_ref[...], preferred_element_type=jnp.float32) + db2_ref[...],
        0.0)
    out_ref[...] = out


@functools.partial(jax.jit, static_argnames=())
def kernel(inp, other_W1, other_b1, other_W2, other_b2,
           self_W1, self_b1, self_W2, self_b2,
           ds_W1, ds_b1, ds_W2, ds_b2,
           Wq_a, Wk_a, Wq_e, Wk_e):
    a_sz = N_ENT * F
    ally_x = inp[:, :a_sz].reshape(B * N_ENT, F)
    enemy_x = inp[:, a_sz:2 * a_sz].reshape(B * N_ENT, F)
    self_x = inp[:, 2 * a_sz:2 * a_sz + F]

    row = lambda v: v.reshape(1, -1)
    wspec = lambda arr: pl.BlockSpec(arr.shape, lambda i: (0,) * arr.ndim)

    weights = [other_W1, row(other_b1), other_W2, row(other_b2),
               self_W1, row(self_b1), self_W2, row(self_b2),
               ds_W1, row(ds_b1), ds_W2, row(ds_b2),
               Wq_a, Wk_a, Wq_e, Wk_e]

    return pl.pallas_call(
        _sampo_kernel,
        grid=(NB,),
        in_specs=[
            pl.BlockSpec((BB * N_ENT, F), lambda i: (i, 0)),
            pl.BlockSpec((BB * N_ENT, F), lambda i: (i, 0)),
            pl.BlockSpec((BB, F), lambda i: (i, 0)),
        ] + [wspec(w) for w in weights],
        out_specs=pl.BlockSpec((BB, DS_OUT), lambda i: (i, 0)),
        out_shape=jax.ShapeDtypeStruct((B, DS_OUT), jnp.float32),
    )(ally_x, enemy_x, self_x, *weights)
